# Initial kernel scaffold; baseline (speedup 1.0000x reference)
import jax
import jax.numpy as jnp
from jax.experimental import pallas as pl


def kernel(x, edge_index, fc0_W, fc0_b, g0_sW, g0_sb, g0_dW, g0_db, g0_tq, g0_bias, g2_sW, g2_sb, g2_dW, g2_db, g2_tq, g2_bias, fc2_W, fc2_b):
    raise NotImplementedError("write your pallas kernel here")


# speedup vs baseline: 5.0976x; 5.0976x over previous
import dataclasses
import functools

import jax
import jax.numpy as jnp
from jax import lax
from jax.experimental import pallas as pl
from jax.experimental.pallas import tpu as pltpu
from jax.experimental.pallas import tpu_sc as plsc

N = 10000
NPAD = 10240
E = 320000
EPAD = 327680
C = 128
K = 128
NCHUNK = 10240 // K
KA = 64
NCHUNK_A = 10240 // KA
NC, NS, L = 2, 16, 16

_mesh = plsc.VectorSubcoreMesh(core_axis_name="c", subcore_axis_name="s")

_cp = pltpu.CompilerParams()
if "needs_layout_passes" in pltpu.CompilerParams.__dataclass_fields__:
    _cp = dataclasses.replace(_cp, needs_layout_passes=False)


def _mm_body(x_ref, w_ref, b_ref, o_ref):
    o_ref[...] = jnp.dot(x_ref[...], w_ref[...],
                         preferred_element_type=jnp.float32) + b_ref[...][None, :]


def _tc_matmul(x, w, b):
    return pl.pallas_call(
        _mm_body,
        out_shape=jax.ShapeDtypeStruct((x.shape[0], w.shape[1]), jnp.float32),
    )(x, w, b)


def _alpha_body(h_ref, al_ref, ar_ref, as_out, ad_out, ws_out, m_out):
    h = h_ref[...]
    a_s = jnp.sum(h * al_ref[...][None, :], axis=1)
    a_d = jnp.sum(h * ar_ref[...][None, :], axis=1)
    ms = jnp.max(a_s) + jnp.max(a_d)
    M = 0.1 * jnp.where(ms >= 0, ms, 0.2 * ms)
    ts = a_s + a_d
    a_self = 0.1 * jnp.where(ts >= 0, ts, 0.2 * ts)
    as_out[...] = a_s
    ad_out[...] = a_d
    ws_out[...] = jnp.exp(a_self - M)
    m_out[...] = jnp.full((L,), M)


def _tc_alphas(h, att_l, att_r):
    return pl.pallas_call(
        _alpha_body,
        out_shape=(
            jax.ShapeDtypeStruct((NPAD,), jnp.float32),
            jax.ShapeDtypeStruct((NPAD,), jnp.float32),
            jax.ShapeDtypeStruct((NPAD,), jnp.float32),
            jax.ShapeDtypeStruct((L,), jnp.float32),
        ),
    )(h, att_l, att_r)


def _fin_relu_body(acc_ref, den_ref, ws_ref, b_ref, o_ref):
    f = acc_ref[0] + acc_ref[1]
    den = jnp.sum(den_ref[...], axis=0) + ws_ref[...]
    o = f / (den + 1e-16)[:, None] + b_ref[...][None, :]
    o_ref[...] = jnp.maximum(o, 0.0)


def _tc_finalize_relu(acc, den, ws, bias):
    return pl.pallas_call(
        _fin_relu_body,
        out_shape=jax.ShapeDtypeStruct((NPAD, C), jnp.float32),
    )(acc, den, ws, bias)


def _fin_mm_body(acc_ref, den_ref, ws_ref, b_ref, w_ref, b2_ref, o_ref):
    f = acc_ref[0] + acc_ref[1]
    den = jnp.sum(den_ref[...], axis=0) + ws_ref[...]
    o = f / (den + 1e-16)[:, None] + b_ref[...][None, :]
    o_ref[...] = jnp.dot(o, w_ref[...],
                         preferred_element_type=jnp.float32) + b2_ref[...][None, :]


def _tc_finalize_matmul(acc, den, ws, bias, w2, b2):
    return pl.pallas_call(
        _fin_mm_body,
        out_shape=jax.ShapeDtypeStruct((NPAD, C), jnp.float32),
    )(acc, den, ws, bias, w2, b2)


def _stats_body(h_hbm, src_hbm, dst_hbm, out_hbm,
                idx_s, idx_d, bufS, bufD, accA, accB):
    cid = lax.axis_index("c")
    sid = lax.axis_index("s")
    wid = cid * NS + sid
    base = wid * (K * NCHUNK)

    zeros = jnp.zeros((L,), jnp.float32)
    for j in range(C // L):
        accA[pl.ds(L * j, L)] = zeros
        accB[pl.ds(L * j, L)] = zeros

    @pl.loop(0, NCHUNK)
    def _chunk(k):
        off = base + k * K
        pltpu.sync_copy(src_hbm.at[pl.ds(off, K)], idx_s)
        pltpu.sync_copy(dst_hbm.at[pl.ds(off, K)], idx_d)
        pltpu.sync_copy(h_hbm.at[idx_s], bufS)
        pltpu.sync_copy(h_hbm.at[idx_d], bufD)

        @pl.loop(0, K)
        def _edge(e):
            for j in range(C // L):
                sl = pl.ds(L * j, L)
                s = bufS[e, sl]
                t = bufD[e, sl]
                df = s - t
                d2 = df * df
                plsc.addupdate(accA.at[sl], d2)
                plsc.addupdate(accB.at[sl], d2 * d2)

    pltpu.sync_copy(accA, out_hbm.at[wid, 0])
    pltpu.sync_copy(accB, out_hbm.at[wid, 1])


_stats_kfn = None


def _sc_stats(h, src, dst):
    global _stats_kfn
    if _stats_kfn is None:
        _stats_kfn = _make_stats_kfn()
    return _stats_kfn(h, src, dst)


def _make_stats_kfn():
    return pl.kernel(
        _stats_body,
        mesh=_mesh,
        out_type=jax.ShapeDtypeStruct((NC * NS, 2, C), jnp.float32),
        scratch_types=[
            pltpu.VMEM((K,), jnp.int32),
            pltpu.VMEM((K,), jnp.int32),
            pltpu.VMEM((K, C), jnp.float32),
            pltpu.VMEM((K, C), jnp.float32),
            pltpu.VMEM((C,), jnp.float32),
            pltpu.VMEM((C,), jnp.float32),
        ],
        compiler_params=_cp,
    )


def _agg_body(h_hbm, src_hbm, dst_hbm, as_hbm, ad_hbm, ws_hbm, m_hbm,
              out_hbm, den_hbm,
              alpha_s, alpha_d, idx_s, idx_d, bufS, wrow,
              wsbuf, m_vmem, denv, acc_sh):
    cid = lax.axis_index("c")
    sid = lax.axis_index("s")
    wid = cid * NS + sid
    base = wid * (KA * NCHUNK_A)
    lane = lax.iota(jnp.int32, L)
    cvec = jnp.full((L,), cid)

    pltpu.sync_copy(as_hbm, alpha_s)
    pltpu.sync_copy(ad_hbm, alpha_d)
    pltpu.sync_copy(m_hbm, m_vmem)
    m = m_vmem[...]

    zeros = jnp.zeros((L,), jnp.float32)

    @pl.loop(0, NPAD // L)
    def _zden(i):
        denv[pl.ds(i * L, L)] = zeros

    row0 = sid * (NPAD // NS)

    @pl.loop(0, (NPAD // NS) // KA)
    def _init(cb):
        r0 = row0 + cb * KA
        pltpu.sync_copy(h_hbm.at[pl.ds(r0, KA)], bufS)
        pltpu.sync_copy(ws_hbm.at[pl.ds(r0, KA)], wsbuf)

        @pl.loop(0, KA // L)
        def _grp(g):
            wv = wsbuf[pl.ds(L * g, L)]
            wv = jnp.where(cvec == 0, wv, 0.0)
            for e in range(L):
                ws = jnp.sum(jnp.where(lane == e, wv, 0.0))
                wb = jnp.full((L,), ws)
                r = L * g + e
                for j in range(C // L):
                    sl = pl.ds(L * j, L)
                    wrow[r, sl] = bufS[r, sl] * wb

        pltpu.sync_copy(wrow, acc_sh.at[pl.ds(r0, KA)])

    plsc.subcore_barrier()

    @pl.loop(0, NCHUNK_A)
    def _chunk(k):
        off = base + k * KA
        pltpu.sync_copy(src_hbm.at[pl.ds(off, KA)], idx_s)
        pltpu.sync_copy(dst_hbm.at[pl.ds(off, KA)], idx_d)
        pltpu.sync_copy(h_hbm.at[idx_s], bufS)

        @pl.loop(0, KA // L)
        def _grp(g):
            sl_g = pl.ds(L * g, L)
            is_v = idx_s[sl_g]
            id_v = idx_d[sl_g]
            as_v = plsc.load_gather(alpha_s, [is_v])
            ad_v = plsc.load_gather(alpha_d, [id_v])
            t = as_v + ad_v
            a = 0.1 * jnp.where(t >= 0, t, 0.2 * t)
            w = jnp.exp(a - m)
            w = jnp.where(is_v != id_v, w, 0.0)
            for e in range(L):
                ws = jnp.sum(jnp.where(lane == e, w, 0.0))
                wb = jnp.full((L,), ws)
                r = L * g + e
                for j in range(C // L):
                    sl = pl.ds(L * j, L)
                    wrow[r, sl] = bufS[r, sl] * wb
            for e in range(L):
                plsc.addupdate_scatter(denv, [id_v], w, mask=lane == e)

        pltpu.sync_copy(wrow, acc_sh.at[idx_d], add=True)

    plsc.subcore_barrier()
    pltpu.sync_copy(acc_sh.at[pl.ds(row0, NPAD // NS)],
                    out_hbm.at[cid, pl.ds(row0, NPAD // NS)])
    pltpu.sync_copy(denv, den_hbm.at[wid])


_agg_kfn = None


def _sc_aggregate(h, src, dst, a_s, a_d, w_self, M):
    global _agg_kfn
    if _agg_kfn is None:
        _agg_kfn = _make_agg_kfn()
    return _agg_kfn(h, src, dst, a_s, a_d, w_self, M)


def _make_agg_kfn():
    return pl.kernel(
        _agg_body,
        mesh=_mesh,
        out_type=(
            jax.ShapeDtypeStruct((NC, NPAD, C), jnp.float32),
            jax.ShapeDtypeStruct((NC * NS, NPAD), jnp.float32),
        ),
        scratch_types=[
            pltpu.VMEM((NPAD,), jnp.float32),
            pltpu.VMEM((NPAD,), jnp.float32),
            pltpu.VMEM((KA,), jnp.int32),
            pltpu.VMEM((KA,), jnp.int32),
            pltpu.VMEM((KA, C), jnp.float32),
            pltpu.VMEM((KA, C), jnp.float32),
            pltpu.VMEM((KA,), jnp.float32),
            pltpu.VMEM((L,), jnp.float32),
            pltpu.VMEM((NPAD,), jnp.float32),
            pltpu.VMEM_SHARED((NPAD, C), jnp.float32),
        ],
        compiler_params=_cp,
    )


def _stats_to_att(parts, sW, sb, dW, db, tq):
    s2 = jnp.sum(parts[:, 0, :], axis=0)
    s4 = jnp.sum(parts[:, 1, :], axis=0)
    m1 = s2 / E
    var = jnp.maximum(s4 - s2 * s2 / E, 0.0) / (E - 1)
    sd = jnp.sqrt(var)
    m2 = sd + 1e-05
    k3 = (m1 * m1 * m1) / (m2 * m2 * m2)
    k4 = (m1 * m1 * m1 * m1) / (m2 * m2 * m2 * m2)
    S = jnp.stack([m1, sd, k3, k4])
    S = jnp.where(jnp.isnan(S), 0.0, S)
    S = jnp.tanh(S)
    nrm = jnp.linalg.norm(S, axis=1, keepdims=True)
    S = (S / jnp.maximum(nrm, 1e-12)).T
    att_l = (S @ sW + sb) @ tq
    att_r = (S @ dW + db) @ tq
    return att_l, att_r


def _spa_layer(h, src, dst, sW, sb, dW, db, tq):
    parts = _sc_stats(h, src, dst)
    att_l, att_r = _stats_to_att(parts, sW, sb, dW, db, tq)
    a_s, a_d, w_self, M = _tc_alphas(h, att_l, att_r)
    acc, den = _sc_aggregate(h, src, dst, a_s, a_d, w_self, M)
    return acc, den, w_self


def kernel(x, edge_index, fc0_W, fc0_b, g0_sW, g0_sb, g0_dW, g0_db, g0_tq,
           g0_bias, g2_sW, g2_sb, g2_dW, g2_db, g2_tq, g2_bias, fc2_W, fc2_b):
    xp = jnp.concatenate([x, jnp.zeros((NPAD - N, C), jnp.float32)], axis=0)
    src = jnp.concatenate(
        [edge_index[0], jnp.zeros((EPAD - E,), edge_index.dtype)])
    dst = jnp.concatenate(
        [edge_index[1], jnp.zeros((EPAD - E,), edge_index.dtype)])

    h = _tc_matmul(xp, fc0_W, fc0_b)
    acc0, den0, ws0 = _spa_layer(h, src, dst, g0_sW, g0_sb, g0_dW, g0_db, g0_tq)
    h2 = _tc_finalize_relu(acc0, den0, ws0, g0_bias)
    acc1, den1, ws1 = _spa_layer(h2, src, dst, g2_sW, g2_sb, g2_dW, g2_db, g2_tq)
    out = _tc_finalize_matmul(acc1, den1, ws1, g2_bias, fc2_W, fc2_b)
    return out[:N]


# speedup vs baseline: 6.6740x; 1.3092x over previous
import dataclasses
import functools

import jax
import jax.numpy as jnp
from jax import lax
from jax.experimental import pallas as pl
from jax.experimental.pallas import tpu as pltpu
from jax.experimental.pallas import tpu_sc as plsc

N = 10000
NPAD = 10240
E = 320000
EPAD = 327680
C = 128
K = 128
NCHUNK = 10240 // K
KA = 64
NCHUNK_A = 10240 // KA
NC, NS, L = 2, 16, 16

_mesh = plsc.VectorSubcoreMesh(core_axis_name="c", subcore_axis_name="s")

_cp = pltpu.CompilerParams()
if "needs_layout_passes" in pltpu.CompilerParams.__dataclass_fields__:
    _cp = dataclasses.replace(_cp, needs_layout_passes=False)


def _mm_body(x_ref, w_ref, b_ref, o_ref):
    o_ref[...] = jnp.dot(x_ref[...], w_ref[...],
                         preferred_element_type=jnp.float32) + b_ref[...][None, :]


def _tc_matmul(x, w, b):
    return pl.pallas_call(
        _mm_body,
        out_shape=jax.ShapeDtypeStruct((x.shape[0], w.shape[1]), jnp.float32),
    )(x, w, b)


def _alpha_body(h_ref, al_ref, ar_ref, as_out, ad_out, ws_out, m_out):
    h = h_ref[...]
    a_s = jnp.sum(h * al_ref[...][None, :], axis=1)
    a_d = jnp.sum(h * ar_ref[...][None, :], axis=1)
    ms = jnp.max(a_s) + jnp.max(a_d)
    M = 0.1 * jnp.where(ms >= 0, ms, 0.2 * ms)
    ts = a_s + a_d
    a_self = 0.1 * jnp.where(ts >= 0, ts, 0.2 * ts)
    as_out[...] = a_s
    ad_out[...] = a_d
    ws_out[...] = jnp.exp(a_self - M)
    m_out[...] = jnp.full((L,), M)


def _tc_alphas(h, att_l, att_r):
    return pl.pallas_call(
        _alpha_body,
        out_shape=(
            jax.ShapeDtypeStruct((NPAD,), jnp.float32),
            jax.ShapeDtypeStruct((NPAD,), jnp.float32),
            jax.ShapeDtypeStruct((NPAD,), jnp.float32),
            jax.ShapeDtypeStruct((L,), jnp.float32),
        ),
    )(h, att_l, att_r)


def _fin_relu_body(acc_ref, den_ref, ws_ref, b_ref, o_ref):
    f = acc_ref[0] + acc_ref[1]
    den = jnp.sum(den_ref[...], axis=0) + ws_ref[...]
    o = f / (den + 1e-16)[:, None] + b_ref[...][None, :]
    o_ref[...] = jnp.maximum(o, 0.0)


def _tc_finalize_relu(acc, den, ws, bias):
    return pl.pallas_call(
        _fin_relu_body,
        out_shape=jax.ShapeDtypeStruct((NPAD, C), jnp.float32),
    )(acc, den, ws, bias)


def _fin_mm_body(acc_ref, den_ref, ws_ref, b_ref, w_ref, b2_ref, o_ref):
    f = acc_ref[0] + acc_ref[1]
    den = jnp.sum(den_ref[...], axis=0) + ws_ref[...]
    o = f / (den + 1e-16)[:, None] + b_ref[...][None, :]
    o_ref[...] = jnp.dot(o, w_ref[...],
                         preferred_element_type=jnp.float32) + b2_ref[...][None, :]


def _tc_finalize_matmul(acc, den, ws, bias, w2, b2):
    return pl.pallas_call(
        _fin_mm_body,
        out_shape=jax.ShapeDtypeStruct((NPAD, C), jnp.float32),
    )(acc, den, ws, bias, w2, b2)


NBUF = 2
EPW = K * NCHUNK


def _stats_body(h_hbm, src_hbm, dst_hbm, out_hbm,
                idx_sv, idx_dv,
                bufS0, bufS1, bufD0, bufD1,
                accA, accB, semS0, semS1, semD0, semD1):
    cid = lax.axis_index("c")
    sid = lax.axis_index("s")
    wid = cid * NS + sid
    base = wid * EPW

    bufS = [bufS0, bufS1]
    bufD = [bufD0, bufD1]
    semS = [semS0, semS1]
    semD = [semD0, semD1]

    pltpu.sync_copy(src_hbm.at[pl.ds(base, EPW)], idx_sv)
    pltpu.sync_copy(dst_hbm.at[pl.ds(base, EPW)], idx_dv)

    zeros = jnp.zeros((L,), jnp.float32)
    for j in range(C // L):
        accA[pl.ds(L * j, L)] = zeros
        accB[pl.ds(L * j, L)] = zeros

    def _fire(c, b):
        sl = pl.ds(c * K, K)
        pltpu.async_copy(h_hbm.at[idx_sv.at[sl]], bufS[b], semS[b])
        pltpu.async_copy(h_hbm.at[idx_dv.at[sl]], bufD[b], semD[b])

    for b in range(NBUF):
        _fire(b, b)

    @pl.loop(0, NCHUNK, step=NBUF)
    def _chunk(k0):
        for b in range(NBUF):
            k = k0 + b
            pltpu.make_async_copy(h_hbm.at[idx_sv.at[pl.ds(0, K)]],
                                  bufS[b], semS[b]).wait()
            pltpu.make_async_copy(h_hbm.at[idx_dv.at[pl.ds(0, K)]],
                                  bufD[b], semD[b]).wait()

            @pl.loop(0, K)
            def _edge(e):
                for j in range(C // L):
                    sl = pl.ds(L * j, L)
                    s = bufS[b][e, sl]
                    t = bufD[b][e, sl]
                    df = s - t
                    d2 = df * df
                    plsc.addupdate(accA.at[sl], d2)
                    plsc.addupdate(accB.at[sl], d2 * d2)

            _fire(lax.rem(k + NBUF, NCHUNK), b)

    for b in range(NBUF):
        pltpu.make_async_copy(h_hbm.at[idx_sv.at[pl.ds(0, K)]],
                              bufS[b], semS[b]).wait()
        pltpu.make_async_copy(h_hbm.at[idx_dv.at[pl.ds(0, K)]],
                              bufD[b], semD[b]).wait()

    pltpu.sync_copy(accA, out_hbm.at[wid, 0])
    pltpu.sync_copy(accB, out_hbm.at[wid, 1])


_stats_kfn = None


def _sc_stats(h, src, dst):
    global _stats_kfn
    if _stats_kfn is None:
        _stats_kfn = _make_stats_kfn()
    return _stats_kfn(h, src, dst)


def _make_stats_kfn():
    return pl.kernel(
        _stats_body,
        mesh=_mesh,
        out_type=jax.ShapeDtypeStruct((NC * NS, 2, C), jnp.float32),
        scratch_types=[
            pltpu.VMEM((EPW,), jnp.int32),
            pltpu.VMEM((EPW,), jnp.int32),
            pltpu.VMEM((K, C), jnp.float32),
            pltpu.VMEM((K, C), jnp.float32),
            pltpu.VMEM((K, C), jnp.float32),
            pltpu.VMEM((K, C), jnp.float32),
            pltpu.VMEM((C,), jnp.float32),
            pltpu.VMEM((C,), jnp.float32),
            pltpu.SemaphoreType.DMA,
            pltpu.SemaphoreType.DMA,
            pltpu.SemaphoreType.DMA,
            pltpu.SemaphoreType.DMA,
        ],
        compiler_params=_cp,
    )


def _agg_body(h_hbm, src_hbm, dst_hbm, as_hbm, ad_hbm, ws_hbm, m_hbm,
              out_hbm, den_hbm,
              alpha_s, alpha_d, idx_s, idx_d, bufS, wrow,
              wsbuf, m_vmem, denv, acc_sh):
    cid = lax.axis_index("c")
    sid = lax.axis_index("s")
    wid = cid * NS + sid
    base = wid * (KA * NCHUNK_A)
    lane = lax.iota(jnp.int32, L)
    cvec = jnp.full((L,), cid)

    pltpu.sync_copy(as_hbm, alpha_s)
    pltpu.sync_copy(ad_hbm, alpha_d)
    pltpu.sync_copy(m_hbm, m_vmem)
    m = m_vmem[...]

    zeros = jnp.zeros((L,), jnp.float32)

    @pl.loop(0, NPAD // L)
    def _zden(i):
        denv[pl.ds(i * L, L)] = zeros

    row0 = sid * (NPAD // NS)

    @pl.loop(0, (NPAD // NS) // KA)
    def _init(cb):
        r0 = row0 + cb * KA
        pltpu.sync_copy(h_hbm.at[pl.ds(r0, KA)], bufS)
        pltpu.sync_copy(ws_hbm.at[pl.ds(r0, KA)], wsbuf)

        @pl.loop(0, KA // L)
        def _grp(g):
            wv = wsbuf[pl.ds(L * g, L)]
            wv = jnp.where(cvec == 0, wv, 0.0)
            for e in range(L):
                ws = jnp.sum(jnp.where(lane == e, wv, 0.0))
                wb = jnp.full((L,), ws)
                r = L * g + e
                for j in range(C // L):
                    sl = pl.ds(L * j, L)
                    wrow[r, sl] = bufS[r, sl] * wb

        pltpu.sync_copy(wrow, acc_sh.at[pl.ds(r0, KA)])

    plsc.subcore_barrier()

    @pl.loop(0, NCHUNK_A)
    def _chunk(k):
        off = base + k * KA
        pltpu.sync_copy(src_hbm.at[pl.ds(off, KA)], idx_s)
        pltpu.sync_copy(dst_hbm.at[pl.ds(off, KA)], idx_d)
        pltpu.sync_copy(h_hbm.at[idx_s], bufS)

        @pl.loop(0, KA // L)
        def _grp(g):
            sl_g = pl.ds(L * g, L)
            is_v = idx_s[sl_g]
            id_v = idx_d[sl_g]
            as_v = plsc.load_gather(alpha_s, [is_v])
            ad_v = plsc.load_gather(alpha_d, [id_v])
            t = as_v + ad_v
            a = 0.1 * jnp.where(t >= 0, t, 0.2 * t)
            w = jnp.exp(a - m)
            w = jnp.where(is_v != id_v, w, 0.0)
            for e in range(L):
                ws = jnp.sum(jnp.where(lane == e, w, 0.0))
                wb = jnp.full((L,), ws)
                r = L * g + e
                for j in range(C // L):
                    sl = pl.ds(L * j, L)
                    wrow[r, sl] = bufS[r, sl] * wb
            for e in range(L):
                plsc.addupdate_scatter(denv, [id_v], w, mask=lane == e)

        pltpu.sync_copy(wrow, acc_sh.at[idx_d], add=True)

    plsc.subcore_barrier()
    pltpu.sync_copy(acc_sh.at[pl.ds(row0, NPAD // NS)],
                    out_hbm.at[cid, pl.ds(row0, NPAD // NS)])
    pltpu.sync_copy(denv, den_hbm.at[wid])


_agg_kfn = None


def _sc_aggregate(h, src, dst, a_s, a_d, w_self, M):
    global _agg_kfn
    if _agg_kfn is None:
        _agg_kfn = _make_agg_kfn()
    return _agg_kfn(h, src, dst, a_s, a_d, w_self, M)


def _make_agg_kfn():
    return pl.kernel(
        _agg_body,
        mesh=_mesh,
        out_type=(
            jax.ShapeDtypeStruct((NC, NPAD, C), jnp.float32),
            jax.ShapeDtypeStruct((NC * NS, NPAD), jnp.float32),
        ),
        scratch_types=[
            pltpu.VMEM((NPAD,), jnp.float32),
            pltpu.VMEM((NPAD,), jnp.float32),
            pltpu.VMEM((KA,), jnp.int32),
            pltpu.VMEM((KA,), jnp.int32),
            pltpu.VMEM((KA, C), jnp.float32),
            pltpu.VMEM((KA, C), jnp.float32),
            pltpu.VMEM((KA,), jnp.float32),
            pltpu.VMEM((L,), jnp.float32),
            pltpu.VMEM((NPAD,), jnp.float32),
            pltpu.VMEM_SHARED((NPAD, C), jnp.float32),
        ],
        compiler_params=_cp,
    )


def _stats_to_att(parts, sW, sb, dW, db, tq):
    s2 = jnp.sum(parts[:, 0, :], axis=0)
    s4 = jnp.sum(parts[:, 1, :], axis=0)
    m1 = s2 / E
    var = jnp.maximum(s4 - s2 * s2 / E, 0.0) / (E - 1)
    sd = jnp.sqrt(var)
    m2 = sd + 1e-05
    k3 = (m1 * m1 * m1) / (m2 * m2 * m2)
    k4 = (m1 * m1 * m1 * m1) / (m2 * m2 * m2 * m2)
    S = jnp.stack([m1, sd, k3, k4])
    S = jnp.where(jnp.isnan(S), 0.0, S)
    S = jnp.tanh(S)
    nrm = jnp.linalg.norm(S, axis=1, keepdims=True)
    S = (S / jnp.maximum(nrm, 1e-12)).T
    att_l = (S @ sW + sb) @ tq
    att_r = (S @ dW + db) @ tq
    return att_l, att_r


def _spa_layer(h, src, dst, sW, sb, dW, db, tq):
    parts = _sc_stats(h, src, dst)
    att_l, att_r = _stats_to_att(parts, sW, sb, dW, db, tq)
    a_s, a_d, w_self, M = _tc_alphas(h, att_l, att_r)
    acc, den = _sc_aggregate(h, src, dst, a_s, a_d, w_self, M)
    return acc, den, w_self


def kernel(x, edge_index, fc0_W, fc0_b, g0_sW, g0_sb, g0_dW, g0_db, g0_tq,
           g0_bias, g2_sW, g2_sb, g2_dW, g2_db, g2_tq, g2_bias, fc2_W, fc2_b):
    xp = jnp.concatenate([x, jnp.zeros((NPAD - N, C), jnp.float32)], axis=0)
    src = jnp.concatenate(
        [edge_index[0], jnp.zeros((EPAD - E,), edge_index.dtype)])
    dst = jnp.concatenate(
        [edge_index[1], jnp.zeros((EPAD - E,), edge_index.dtype)])

    h = _tc_matmul(xp, fc0_W, fc0_b)
    acc0, den0, ws0 = _spa_layer(h, src, dst, g0_sW, g0_sb, g0_dW, g0_db, g0_tq)
    h2 = _tc_finalize_relu(acc0, den0, ws0, g0_bias)
    acc1, den1, ws1 = _spa_layer(h2, src, dst, g2_sW, g2_sb, g2_dW, g2_db, g2_tq)
    out = _tc_finalize_matmul(acc1, den1, ws1, g2_bias, fc2_W, fc2_b)
    return out[:N]


# speedup vs baseline: 7.0839x; 1.0614x over previous
import dataclasses
import functools

import jax
import jax.numpy as jnp
from jax import lax
from jax.experimental import pallas as pl
from jax.experimental.pallas import tpu as pltpu
from jax.experimental.pallas import tpu_sc as plsc

N = 10000
NPAD = 10240
E = 320000
EPAD = 327680
C = 128
K = 128
NCHUNK = 10240 // K
KA = 32
NCHUNK_A = 10240 // KA
NC, NS, L = 2, 16, 16

_mesh = plsc.VectorSubcoreMesh(core_axis_name="c", subcore_axis_name="s")

_cp = pltpu.CompilerParams()
if "needs_layout_passes" in pltpu.CompilerParams.__dataclass_fields__:
    _cp = dataclasses.replace(_cp, needs_layout_passes=False)


def _mm_body(x_ref, w_ref, b_ref, o_ref):
    o_ref[...] = jnp.dot(x_ref[...], w_ref[...],
                         preferred_element_type=jnp.float32) + b_ref[...][None, :]


def _tc_matmul(x, w, b):
    return pl.pallas_call(
        _mm_body,
        out_shape=jax.ShapeDtypeStruct((x.shape[0], w.shape[1]), jnp.float32),
    )(x, w, b)


def _alpha_body(h_ref, al_ref, ar_ref, as_out, ad_out, ws_out, m_out):
    h = h_ref[...]
    a_s = jnp.sum(h * al_ref[...][None, :], axis=1)
    a_d = jnp.sum(h * ar_ref[...][None, :], axis=1)
    ms = jnp.max(a_s) + jnp.max(a_d)
    M = 0.1 * jnp.where(ms >= 0, ms, 0.2 * ms)
    ts = a_s + a_d
    a_self = 0.1 * jnp.where(ts >= 0, ts, 0.2 * ts)
    as_out[...] = a_s
    ad_out[...] = a_d
    ws_out[...] = jnp.exp(a_self - M)
    m_out[...] = jnp.full((L,), M)


def _tc_alphas(h, att_l, att_r):
    return pl.pallas_call(
        _alpha_body,
        out_shape=(
            jax.ShapeDtypeStruct((NPAD,), jnp.float32),
            jax.ShapeDtypeStruct((NPAD,), jnp.float32),
            jax.ShapeDtypeStruct((NPAD,), jnp.float32),
            jax.ShapeDtypeStruct((L,), jnp.float32),
        ),
    )(h, att_l, att_r)


def _fin_relu_body(acc_ref, den_ref, ws_ref, b_ref, o_ref):
    f = acc_ref[0] + acc_ref[1]
    den = jnp.sum(den_ref[...], axis=0) + ws_ref[...]
    o = f / (den + 1e-16)[:, None] + b_ref[...][None, :]
    o_ref[...] = jnp.maximum(o, 0.0)


def _tc_finalize_relu(acc, den, ws, bias):
    return pl.pallas_call(
        _fin_relu_body,
        out_shape=jax.ShapeDtypeStruct((NPAD, C), jnp.float32),
    )(acc, den, ws, bias)


def _fin_mm_body(acc_ref, den_ref, ws_ref, b_ref, w_ref, b2_ref, o_ref):
    f = acc_ref[0] + acc_ref[1]
    den = jnp.sum(den_ref[...], axis=0) + ws_ref[...]
    o = f / (den + 1e-16)[:, None] + b_ref[...][None, :]
    o_ref[...] = jnp.dot(o, w_ref[...],
                         preferred_element_type=jnp.float32) + b2_ref[...][None, :]


def _tc_finalize_matmul(acc, den, ws, bias, w2, b2):
    return pl.pallas_call(
        _fin_mm_body,
        out_shape=jax.ShapeDtypeStruct((NPAD, C), jnp.float32),
    )(acc, den, ws, bias, w2, b2)


NBUF = 2
EPW = K * NCHUNK


def _stats_body(h_hbm, src_hbm, dst_hbm, out_hbm,
                idx_sv, idx_dv,
                bufS0, bufS1, bufD0, bufD1,
                accA, accB, semS0, semS1, semD0, semD1):
    cid = lax.axis_index("c")
    sid = lax.axis_index("s")
    wid = cid * NS + sid
    base = wid * EPW

    bufS = [bufS0, bufS1]
    bufD = [bufD0, bufD1]
    semS = [semS0, semS1]
    semD = [semD0, semD1]

    pltpu.sync_copy(src_hbm.at[pl.ds(base, EPW)], idx_sv)
    pltpu.sync_copy(dst_hbm.at[pl.ds(base, EPW)], idx_dv)

    zeros = jnp.zeros((L,), jnp.float32)
    for j in range(C // L):
        accA[pl.ds(L * j, L)] = zeros
        accB[pl.ds(L * j, L)] = zeros

    def _fire(c, b):
        sl = pl.ds(c * K, K)
        pltpu.async_copy(h_hbm.at[idx_sv.at[sl]], bufS[b], semS[b])
        pltpu.async_copy(h_hbm.at[idx_dv.at[sl]], bufD[b], semD[b])

    for b in range(NBUF):
        _fire(b, b)

    @pl.loop(0, NCHUNK, step=NBUF)
    def _chunk(k0):
        for b in range(NBUF):
            k = k0 + b
            pltpu.make_async_copy(h_hbm.at[idx_sv.at[pl.ds(0, K)]],
                                  bufS[b], semS[b]).wait()
            pltpu.make_async_copy(h_hbm.at[idx_dv.at[pl.ds(0, K)]],
                                  bufD[b], semD[b]).wait()

            @pl.loop(0, K)
            def _edge(e):
                for j in range(C // L):
                    sl = pl.ds(L * j, L)
                    s = bufS[b][e, sl]
                    t = bufD[b][e, sl]
                    df = s - t
                    d2 = df * df
                    plsc.addupdate(accA.at[sl], d2)
                    plsc.addupdate(accB.at[sl], d2 * d2)

            _fire(lax.rem(k + NBUF, NCHUNK), b)

    for b in range(NBUF):
        pltpu.make_async_copy(h_hbm.at[idx_sv.at[pl.ds(0, K)]],
                              bufS[b], semS[b]).wait()
        pltpu.make_async_copy(h_hbm.at[idx_dv.at[pl.ds(0, K)]],
                              bufD[b], semD[b]).wait()

    pltpu.sync_copy(accA, out_hbm.at[wid, 0])
    pltpu.sync_copy(accB, out_hbm.at[wid, 1])


_stats_kfn = None


def _sc_stats(h, src, dst):
    global _stats_kfn
    if _stats_kfn is None:
        _stats_kfn = _make_stats_kfn()
    return _stats_kfn(h, src, dst)


def _make_stats_kfn():
    return pl.kernel(
        _stats_body,
        mesh=_mesh,
        out_type=jax.ShapeDtypeStruct((NC * NS, 2, C), jnp.float32),
        scratch_types=[
            pltpu.VMEM((EPW,), jnp.int32),
            pltpu.VMEM((EPW,), jnp.int32),
            pltpu.VMEM((K, C), jnp.float32),
            pltpu.VMEM((K, C), jnp.float32),
            pltpu.VMEM((K, C), jnp.float32),
            pltpu.VMEM((K, C), jnp.float32),
            pltpu.VMEM((C,), jnp.float32),
            pltpu.VMEM((C,), jnp.float32),
            pltpu.SemaphoreType.DMA,
            pltpu.SemaphoreType.DMA,
            pltpu.SemaphoreType.DMA,
            pltpu.SemaphoreType.DMA,
        ],
        compiler_params=_cp,
    )


def _agg_body(h_hbm, pack_hbm, as_hbm, ad_hbm, ws_hbm, m_hbm,
              out_hbm, den_hbm,
              alpha_s, alpha_d, idx0, idx1, bufS0, bufS1, wrow,
              wsbuf, m_vmem, denv, acc_sh,
              semI0, semI1, semG0, semG1):
    cid = lax.axis_index("c")
    sid = lax.axis_index("s")
    wid = cid * NS + sid
    cbase = wid * NCHUNK_A
    lane = lax.iota(jnp.int32, L)
    cvec = jnp.full((L,), cid)

    idx2 = [idx0, idx1]
    bufS = [bufS0, bufS1]
    semI = [semI0, semI1]
    semG = [semG0, semG1]

    pltpu.sync_copy(pack_hbm.at[cbase], idx2[0])
    pltpu.async_copy(pack_hbm.at[cbase + 1], idx2[1], semI[1])
    pltpu.async_copy(h_hbm.at[idx2[0].at[0]], bufS[0], semG[0])

    pltpu.sync_copy(as_hbm, alpha_s)
    pltpu.sync_copy(ad_hbm, alpha_d)
    pltpu.sync_copy(m_hbm, m_vmem)
    m = m_vmem[...]

    zeros = jnp.zeros((L,), jnp.float32)

    @pl.loop(0, NPAD // L)
    def _zden(i):
        denv[pl.ds(i * L, L)] = zeros

    row0 = sid * (NPAD // NS)

    @pl.loop(0, (NPAD // NS) // KA)
    def _init(cb):
        r0 = row0 + cb * KA
        pltpu.sync_copy(h_hbm.at[pl.ds(r0, KA)], bufS1)
        pltpu.sync_copy(ws_hbm.at[pl.ds(r0, KA)], wsbuf)

        @pl.loop(0, KA // L)
        def _grp(g):
            wv = wsbuf[pl.ds(L * g, L)]
            wv = jnp.where(cvec == 0, wv, 0.0)
            for e in range(L):
                ws = jnp.sum(jnp.where(lane == e, wv, 0.0))
                wb = jnp.full((L,), ws)
                r = L * g + e
                for j in range(C // L):
                    sl = pl.ds(L * j, L)
                    wrow[r, sl] = bufS1[r, sl] * wb

        pltpu.sync_copy(wrow, acc_sh.at[pl.ds(r0, KA)])

    plsc.subcore_barrier()

    @pl.loop(0, NCHUNK_A, step=NBUF)
    def _chunk(k0):
        for b in range(NBUF):
            k = k0 + b
            bn = (b + 1) % NBUF
            pltpu.make_async_copy(h_hbm.at[idx2[b].at[0]],
                                  bufS[b], semG[b]).wait()
            @pl.loop(0, KA // L)
            def _grp(g):
                sl_g = pl.ds(L * g, L)
                is_v = idx2[b][0, sl_g]
                id_v = idx2[b][1, sl_g]
                as_v = plsc.load_gather(alpha_s, [is_v])
                ad_v = plsc.load_gather(alpha_d, [id_v])
                t = as_v + ad_v
                a = 0.1 * jnp.where(t >= 0, t, 0.2 * t)
                w = jnp.exp(a - m)
                w = jnp.where(is_v != id_v, w, 0.0)
                for e in range(L):
                    ws = jnp.sum(jnp.where(lane == e, w, 0.0))
                    wb = jnp.full((L,), ws)
                    r = L * g + e
                    for j in range(C // L):
                        sl = pl.ds(L * j, L)
                        wrow[r, sl] = bufS[b][r, sl] * wb
                for e in range(L):
                    plsc.addupdate_scatter(denv, [id_v], w, mask=lane == e)

            pltpu.sync_copy(wrow, acc_sh.at[idx2[b].at[1]], add=True)
            pltpu.async_copy(pack_hbm.at[cbase + lax.rem(k + 2, NCHUNK_A)],
                             idx2[b], semI[b])
            pltpu.make_async_copy(pack_hbm.at[cbase], idx2[bn],
                                  semI[bn]).wait()
            pltpu.async_copy(h_hbm.at[idx2[bn].at[0]], bufS[bn], semG[bn])

    pltpu.make_async_copy(pack_hbm.at[cbase], idx2[1], semI[1]).wait()
    pltpu.make_async_copy(h_hbm.at[idx2[0].at[0]], bufS[0], semG[0]).wait()

    plsc.subcore_barrier()
    pltpu.sync_copy(acc_sh.at[pl.ds(row0, NPAD // NS)],
                    out_hbm.at[cid, pl.ds(row0, NPAD // NS)])
    pltpu.sync_copy(denv, den_hbm.at[wid])


_agg_kfn = None


def _sc_aggregate(h, pack, a_s, a_d, w_self, M):
    global _agg_kfn
    if _agg_kfn is None:
        _agg_kfn = _make_agg_kfn()
    return _agg_kfn(h, pack, a_s, a_d, w_self, M)


def _make_agg_kfn():
    return pl.kernel(
        _agg_body,
        mesh=_mesh,
        out_type=(
            jax.ShapeDtypeStruct((NC, NPAD, C), jnp.float32),
            jax.ShapeDtypeStruct((NC * NS, NPAD), jnp.float32),
        ),
        scratch_types=[
            pltpu.VMEM((NPAD,), jnp.float32),
            pltpu.VMEM((NPAD,), jnp.float32),
            pltpu.VMEM((2, KA), jnp.int32),
            pltpu.VMEM((2, KA), jnp.int32),
            pltpu.VMEM((KA, C), jnp.float32),
            pltpu.VMEM((KA, C), jnp.float32),
            pltpu.VMEM((KA, C), jnp.float32),
            pltpu.VMEM((KA,), jnp.float32),
            pltpu.VMEM((L,), jnp.float32),
            pltpu.VMEM((NPAD,), jnp.float32),
            pltpu.VMEM_SHARED((NPAD, C), jnp.float32),
            pltpu.SemaphoreType.DMA,
            pltpu.SemaphoreType.DMA,
            pltpu.SemaphoreType.DMA,
            pltpu.SemaphoreType.DMA,
        ],
        compiler_params=_cp,
    )


def _stats_to_att(parts, sW, sb, dW, db, tq):
    s2 = jnp.sum(parts[:, 0, :], axis=0)
    s4 = jnp.sum(parts[:, 1, :], axis=0)
    m1 = s2 / E
    var = jnp.maximum(s4 - s2 * s2 / E, 0.0) / (E - 1)
    sd = jnp.sqrt(var)
    m2 = sd + 1e-05
    k3 = (m1 * m1 * m1) / (m2 * m2 * m2)
    k4 = (m1 * m1 * m1 * m1) / (m2 * m2 * m2 * m2)
    S = jnp.stack([m1, sd, k3, k4])
    S = jnp.where(jnp.isnan(S), 0.0, S)
    S = jnp.tanh(S)
    nrm = jnp.linalg.norm(S, axis=1, keepdims=True)
    S = (S / jnp.maximum(nrm, 1e-12)).T
    att_l = (S @ sW + sb) @ tq
    att_r = (S @ dW + db) @ tq
    return att_l, att_r


def _spa_layer(h, src, dst, pack, sW, sb, dW, db, tq):
    parts = _sc_stats(h, src, dst)
    att_l, att_r = _stats_to_att(parts, sW, sb, dW, db, tq)
    a_s, a_d, w_self, M = _tc_alphas(h, att_l, att_r)
    acc, den = _sc_aggregate(h, pack, a_s, a_d, w_self, M)
    return acc, den, w_self


def kernel(x, edge_index, fc0_W, fc0_b, g0_sW, g0_sb, g0_dW, g0_db, g0_tq,
           g0_bias, g2_sW, g2_sb, g2_dW, g2_db, g2_tq, g2_bias, fc2_W, fc2_b):
    xp = jnp.concatenate([x, jnp.zeros((NPAD - N, C), jnp.float32)], axis=0)
    src = jnp.concatenate(
        [edge_index[0], jnp.zeros((EPAD - E,), edge_index.dtype)])
    dst = jnp.concatenate(
        [edge_index[1], jnp.zeros((EPAD - E,), edge_index.dtype)])
    pack = jnp.stack([src.reshape(EPAD // KA, KA),
                      dst.reshape(EPAD // KA, KA)], axis=1)

    h = _tc_matmul(xp, fc0_W, fc0_b)
    acc0, den0, ws0 = _spa_layer(h, src, dst, pack,
                                 g0_sW, g0_sb, g0_dW, g0_db, g0_tq)
    h2 = _tc_finalize_relu(acc0, den0, ws0, g0_bias)
    acc1, den1, ws1 = _spa_layer(h2, src, dst, pack,
                                 g2_sW, g2_sb, g2_dW, g2_db, g2_tq)
    out = _tc_finalize_matmul(acc1, den1, ws1, g2_bias, fc2_W, fc2_b)
    return out[:N]


# speedup vs baseline: 7.0841x; 1.0000x over previous
import dataclasses
import functools

import jax
import jax.numpy as jnp
from jax import lax
from jax.experimental import pallas as pl
from jax.experimental.pallas import tpu as pltpu
from jax.experimental.pallas import tpu_sc as plsc

N = 10000
NPAD = 10240
E = 320000
EPAD = 327680
C = 128
K = 128
NCHUNK = 10240 // K
KA = 32
NCHUNK_A = 10240 // KA
NC, NS, L = 2, 16, 16

_mesh = plsc.VectorSubcoreMesh(core_axis_name="c", subcore_axis_name="s")

_cp = pltpu.CompilerParams()
if "needs_layout_passes" in pltpu.CompilerParams.__dataclass_fields__:
    _cp = dataclasses.replace(_cp, needs_layout_passes=False)


def _mm_body(x_ref, w_ref, b_ref, o_ref):
    o_ref[...] = jnp.dot(x_ref[...], w_ref[...],
                         preferred_element_type=jnp.float32) + b_ref[...][None, :]


def _tc_matmul(x, w, b):
    return pl.pallas_call(
        _mm_body,
        out_shape=jax.ShapeDtypeStruct((x.shape[0], w.shape[1]), jnp.float32),
    )(x, w, b)


def _alpha_body(h_ref, al_ref, ar_ref, as_out, ad_out, ws_out, m_out):
    h = h_ref[...]
    a_s = jnp.sum(h * al_ref[...][None, :], axis=1)
    a_d = jnp.sum(h * ar_ref[...][None, :], axis=1)
    ms = jnp.max(a_s) + jnp.max(a_d)
    M = 0.1 * jnp.where(ms >= 0, ms, 0.2 * ms)
    ts = a_s + a_d
    a_self = 0.1 * jnp.where(ts >= 0, ts, 0.2 * ts)
    as_out[...] = a_s
    ad_out[...] = a_d
    ws_out[...] = jnp.exp(a_self - M)
    m_out[...] = jnp.full((L,), M)


def _tc_alphas(h, att_l, att_r):
    return pl.pallas_call(
        _alpha_body,
        out_shape=(
            jax.ShapeDtypeStruct((NPAD,), jnp.float32),
            jax.ShapeDtypeStruct((NPAD,), jnp.float32),
            jax.ShapeDtypeStruct((NPAD,), jnp.float32),
            jax.ShapeDtypeStruct((L,), jnp.float32),
        ),
    )(h, att_l, att_r)


def _fin_relu_body(acc_ref, den_ref, ws_ref, b_ref, o_ref):
    f = acc_ref[0] + acc_ref[1]
    den = jnp.sum(den_ref[...], axis=0) + ws_ref[...]
    o = f / (den + 1e-16)[:, None] + b_ref[...][None, :]
    o_ref[...] = jnp.maximum(o, 0.0)


def _tc_finalize_relu(acc, den, ws, bias):
    return pl.pallas_call(
        _fin_relu_body,
        out_shape=jax.ShapeDtypeStruct((NPAD, C), jnp.float32),
    )(acc, den, ws, bias)


def _fin_mm_body(acc_ref, den_ref, ws_ref, b_ref, w_ref, b2_ref, o_ref):
    f = acc_ref[0] + acc_ref[1]
    den = jnp.sum(den_ref[...], axis=0) + ws_ref[...]
    o = f / (den + 1e-16)[:, None] + b_ref[...][None, :]
    o_ref[...] = jnp.dot(o, w_ref[...],
                         preferred_element_type=jnp.float32) + b2_ref[...][None, :]


def _tc_finalize_matmul(acc, den, ws, bias, w2, b2):
    return pl.pallas_call(
        _fin_mm_body,
        out_shape=jax.ShapeDtypeStruct((NPAD, C), jnp.float32),
    )(acc, den, ws, bias, w2, b2)


NBUF = 2
EPW = K * NCHUNK


def _stats_body(h_hbm, src_hbm, dst_hbm, out_hbm,
                idx_sv, idx_dv,
                bufS0, bufS1, bufD0, bufD1,
                accA, accB, semS0, semS1, semD0, semD1):
    cid = lax.axis_index("c")
    sid = lax.axis_index("s")
    wid = cid * NS + sid
    base = wid * EPW

    bufS = [bufS0, bufS1]
    bufD = [bufD0, bufD1]
    semS = [semS0, semS1]
    semD = [semD0, semD1]

    pltpu.sync_copy(src_hbm.at[pl.ds(base, EPW)], idx_sv)
    pltpu.sync_copy(dst_hbm.at[pl.ds(base, EPW)], idx_dv)

    zeros = jnp.zeros((L,), jnp.float32)
    for j in range(C // L):
        accA[pl.ds(L * j, L)] = zeros
        accB[pl.ds(L * j, L)] = zeros

    def _fire(c, b):
        sl = pl.ds(c * K, K)
        pltpu.async_copy(h_hbm.at[idx_sv.at[sl]], bufS[b], semS[b])
        pltpu.async_copy(h_hbm.at[idx_dv.at[sl]], bufD[b], semD[b])

    for b in range(NBUF):
        _fire(b, b)

    @pl.loop(0, NCHUNK, step=NBUF)
    def _chunk(k0):
        for b in range(NBUF):
            k = k0 + b
            pltpu.make_async_copy(h_hbm.at[idx_sv.at[pl.ds(0, K)]],
                                  bufS[b], semS[b]).wait()
            pltpu.make_async_copy(h_hbm.at[idx_dv.at[pl.ds(0, K)]],
                                  bufD[b], semD[b]).wait()

            @pl.loop(0, K)
            def _edge(e):
                for j in range(C // L):
                    sl = pl.ds(L * j, L)
                    s = bufS[b][e, sl].astype(jnp.float32)
                    t = bufD[b][e, sl].astype(jnp.float32)
                    df = s - t
                    d2 = df * df
                    plsc.addupdate(accA.at[sl], d2)
                    plsc.addupdate(accB.at[sl], d2 * d2)

            _fire(lax.rem(k + NBUF, NCHUNK), b)

    for b in range(NBUF):
        pltpu.make_async_copy(h_hbm.at[idx_sv.at[pl.ds(0, K)]],
                              bufS[b], semS[b]).wait()
        pltpu.make_async_copy(h_hbm.at[idx_dv.at[pl.ds(0, K)]],
                              bufD[b], semD[b]).wait()

    pltpu.sync_copy(accA, out_hbm.at[wid, 0])
    pltpu.sync_copy(accB, out_hbm.at[wid, 1])


_stats_kfn = None


def _sc_stats(h, src, dst):
    global _stats_kfn
    if _stats_kfn is None:
        _stats_kfn = _make_stats_kfn()
    return _stats_kfn(h, src, dst)


def _make_stats_kfn():
    return pl.kernel(
        _stats_body,
        mesh=_mesh,
        out_type=jax.ShapeDtypeStruct((NC * NS, 2, C), jnp.float32),
        scratch_types=[
            pltpu.VMEM((EPW,), jnp.int32),
            pltpu.VMEM((EPW,), jnp.int32),
            pltpu.VMEM((K, C), jnp.float32),
            pltpu.VMEM((K, C), jnp.float32),
            pltpu.VMEM((K, C), jnp.float32),
            pltpu.VMEM((K, C), jnp.float32),
            pltpu.VMEM((C,), jnp.float32),
            pltpu.VMEM((C,), jnp.float32),
            pltpu.SemaphoreType.DMA,
            pltpu.SemaphoreType.DMA,
            pltpu.SemaphoreType.DMA,
            pltpu.SemaphoreType.DMA,
        ],
        compiler_params=_cp,
    )


def _agg_body(h_hbm, pack_hbm, as_hbm, ad_hbm, ws_hbm, m_hbm,
              out_hbm, den_hbm,
              alpha_s, alpha_d, idx0, idx1, bufS0, bufS1, wrow,
              wsbuf, m_vmem, denv, acc_sh,
              semI0, semI1, semG0, semG1):
    cid = lax.axis_index("c")
    sid = lax.axis_index("s")
    wid = cid * NS + sid
    cbase = wid * NCHUNK_A
    lane = lax.iota(jnp.int32, L)
    cvec = jnp.full((L,), cid)

    idx2 = [idx0, idx1]
    bufS = [bufS0, bufS1]
    semI = [semI0, semI1]
    semG = [semG0, semG1]

    pltpu.sync_copy(pack_hbm.at[cbase], idx2[0])
    pltpu.async_copy(pack_hbm.at[cbase + 1], idx2[1], semI[1])
    pltpu.async_copy(h_hbm.at[idx2[0].at[0]], bufS[0], semG[0])

    pltpu.sync_copy(as_hbm, alpha_s)
    pltpu.sync_copy(ad_hbm, alpha_d)
    pltpu.sync_copy(m_hbm, m_vmem)
    m = m_vmem[...]

    zeros = jnp.zeros((L,), jnp.float32)

    @pl.loop(0, NPAD // L)
    def _zden(i):
        denv[pl.ds(i * L, L)] = zeros

    row0 = sid * (NPAD // NS)

    @pl.loop(0, (NPAD // NS) // KA)
    def _init(cb):
        r0 = row0 + cb * KA
        pltpu.sync_copy(h_hbm.at[pl.ds(r0, KA)], bufS1)
        pltpu.sync_copy(ws_hbm.at[pl.ds(r0, KA)], wsbuf)

        @pl.loop(0, KA // L)
        def _grp(g):
            wv = wsbuf[pl.ds(L * g, L)]
            wv = jnp.where(cvec == 0, wv, 0.0)
            for e in range(L):
                ws = jnp.sum(jnp.where(lane == e, wv, 0.0))
                wb = jnp.full((L,), ws)
                r = L * g + e
                for j in range(C // L):
                    sl = pl.ds(L * j, L)
                    wrow[r, sl] = bufS1[r, sl] * wb

        pltpu.sync_copy(wrow, acc_sh.at[pl.ds(r0, KA)])

    plsc.subcore_barrier()

    @pl.loop(0, NCHUNK_A, step=NBUF)
    def _chunk(k0):
        for b in range(NBUF):
            k = k0 + b
            bn = (b + 1) % NBUF
            pltpu.make_async_copy(h_hbm.at[idx2[b].at[0]],
                                  bufS[b], semG[b]).wait()
            @pl.loop(0, KA // L)
            def _grp(g):
                sl_g = pl.ds(L * g, L)
                is_v = idx2[b][0, sl_g]
                id_v = idx2[b][1, sl_g]
                as_v = plsc.load_gather(alpha_s, [is_v])
                ad_v = plsc.load_gather(alpha_d, [id_v])
                t = as_v + ad_v
                a = 0.1 * jnp.where(t >= 0, t, 0.2 * t)
                w = jnp.exp(a - m)
                w = jnp.where(is_v != id_v, w, 0.0)
                for e in range(L):
                    ws = jnp.sum(jnp.where(lane == e, w, 0.0))
                    wb = jnp.full((L,), ws)
                    r = L * g + e
                    for j in range(C // L):
                        sl = pl.ds(L * j, L)
                        wrow[r, sl] = bufS[b][r, sl] * wb
                for e in range(L):
                    plsc.addupdate_scatter(denv, [id_v], w, mask=lane == e)

            pltpu.sync_copy(wrow, acc_sh.at[idx2[b].at[1]], add=True)
            pltpu.async_copy(pack_hbm.at[cbase + lax.rem(k + 2, NCHUNK_A)],
                             idx2[b], semI[b])
            pltpu.make_async_copy(pack_hbm.at[cbase], idx2[bn],
                                  semI[bn]).wait()
            pltpu.async_copy(h_hbm.at[idx2[bn].at[0]], bufS[bn], semG[bn])

    pltpu.make_async_copy(pack_hbm.at[cbase], idx2[1], semI[1]).wait()
    pltpu.make_async_copy(h_hbm.at[idx2[0].at[0]], bufS[0], semG[0]).wait()

    plsc.subcore_barrier()
    pltpu.sync_copy(acc_sh.at[pl.ds(row0, NPAD // NS)],
                    out_hbm.at[cid, pl.ds(row0, NPAD // NS)])
    pltpu.sync_copy(denv, den_hbm.at[wid])


_agg_kfn = None


def _sc_aggregate(h, pack, a_s, a_d, w_self, M):
    global _agg_kfn
    if _agg_kfn is None:
        _agg_kfn = _make_agg_kfn()
    return _agg_kfn(h, pack, a_s, a_d, w_self, M)


def _make_agg_kfn():
    return pl.kernel(
        _agg_body,
        mesh=_mesh,
        out_type=(
            jax.ShapeDtypeStruct((NC, NPAD, C), jnp.float32),
            jax.ShapeDtypeStruct((NC * NS, NPAD), jnp.float32),
        ),
        scratch_types=[
            pltpu.VMEM((NPAD,), jnp.float32),
            pltpu.VMEM((NPAD,), jnp.float32),
            pltpu.VMEM((2, KA), jnp.int32),
            pltpu.VMEM((2, KA), jnp.int32),
            pltpu.VMEM((KA, C), jnp.float32),
            pltpu.VMEM((KA, C), jnp.float32),
            pltpu.VMEM((KA, C), jnp.float32),
            pltpu.VMEM((KA,), jnp.float32),
            pltpu.VMEM((L,), jnp.float32),
            pltpu.VMEM((NPAD,), jnp.float32),
            pltpu.VMEM_SHARED((NPAD, C), jnp.float32),
            pltpu.SemaphoreType.DMA,
            pltpu.SemaphoreType.DMA,
            pltpu.SemaphoreType.DMA,
            pltpu.SemaphoreType.DMA,
        ],
        compiler_params=_cp,
    )


def _stats_to_att(parts, sW, sb, dW, db, tq):
    s2 = jnp.sum(parts[:, 0, :], axis=0)
    s4 = jnp.sum(parts[:, 1, :], axis=0)
    m1 = s2 / E
    var = jnp.maximum(s4 - s2 * s2 / E, 0.0) / (E - 1)
    sd = jnp.sqrt(var)
    m2 = sd + 1e-05
    k3 = (m1 * m1 * m1) / (m2 * m2 * m2)
    k4 = (m1 * m1 * m1 * m1) / (m2 * m2 * m2 * m2)
    S = jnp.stack([m1, sd, k3, k4])
    S = jnp.where(jnp.isnan(S), 0.0, S)
    S = jnp.tanh(S)
    nrm = jnp.linalg.norm(S, axis=1, keepdims=True)
    S = (S / jnp.maximum(nrm, 1e-12)).T
    att_l = (S @ sW + sb) @ tq
    att_r = (S @ dW + db) @ tq
    return att_l, att_r


def _spa_layer(h, src, dst, pack, sW, sb, dW, db, tq):
    parts = _sc_stats(h, src, dst)
    att_l, att_r = _stats_to_att(parts, sW, sb, dW, db, tq)
    a_s, a_d, w_self, M = _tc_alphas(h, att_l, att_r)
    acc, den = _sc_aggregate(h, pack, a_s, a_d, w_self, M)
    return acc, den, w_self


def kernel(x, edge_index, fc0_W, fc0_b, g0_sW, g0_sb, g0_dW, g0_db, g0_tq,
           g0_bias, g2_sW, g2_sb, g2_dW, g2_db, g2_tq, g2_bias, fc2_W, fc2_b):
    xp = jnp.concatenate([x, jnp.zeros((NPAD - N, C), jnp.float32)], axis=0)
    src = jnp.concatenate(
        [edge_index[0], jnp.zeros((EPAD - E,), edge_index.dtype)])
    dst = jnp.concatenate(
        [edge_index[1], jnp.zeros((EPAD - E,), edge_index.dtype)])
    pack = jnp.stack([src.reshape(EPAD // KA, KA),
                      dst.reshape(EPAD // KA, KA)], axis=1)

    h = _tc_matmul(xp, fc0_W, fc0_b)
    acc0, den0, ws0 = _spa_layer(h, src, dst, pack,
                                 g0_sW, g0_sb, g0_dW, g0_db, g0_tq)
    h2 = _tc_finalize_relu(acc0, den0, ws0, g0_bias)
    acc1, den1, ws1 = _spa_layer(h2, src, dst, pack,
                                 g2_sW, g2_sb, g2_dW, g2_db, g2_tq)
    out = _tc_finalize_matmul(acc1, den1, ws1, g2_bias, fc2_W, fc2_b)
    return out[:N]


# speedup vs baseline: 7.0851x; 1.0001x over previous
import dataclasses
import functools

import jax
import jax.numpy as jnp
from jax import lax
from jax.experimental import pallas as pl
from jax.experimental.pallas import tpu as pltpu
from jax.experimental.pallas import tpu_sc as plsc

N = 10000
NPAD = 10240
E = 320000
EPAD = 327680
C = 128
K = 128
NCHUNK = 10240 // K
KA = 32
NCHUNK_A = 10240 // KA
NC, NS, L = 2, 16, 16

_mesh = plsc.VectorSubcoreMesh(core_axis_name="c", subcore_axis_name="s")

_cp = pltpu.CompilerParams()
if "needs_layout_passes" in pltpu.CompilerParams.__dataclass_fields__:
    _cp = dataclasses.replace(_cp, needs_layout_passes=False)


def _mm_body(x_ref, w_ref, b_ref, o_ref):
    o_ref[...] = jnp.dot(x_ref[...], w_ref[...],
                         preferred_element_type=jnp.float32) + b_ref[...][None, :]


def _tc_matmul(x, w, b):
    return pl.pallas_call(
        _mm_body,
        out_shape=jax.ShapeDtypeStruct((x.shape[0], w.shape[1]), jnp.float32),
    )(x, w, b)


def _alpha_body(h_ref, al_ref, ar_ref, as_out, ad_out, ws_out, m_out):
    h = h_ref[...]
    a_s = jnp.sum(h * al_ref[...][None, :], axis=1)
    a_d = jnp.sum(h * ar_ref[...][None, :], axis=1)
    ms = jnp.max(a_s) + jnp.max(a_d)
    M = 0.1 * jnp.where(ms >= 0, ms, 0.2 * ms)
    ts = a_s + a_d
    a_self = 0.1 * jnp.where(ts >= 0, ts, 0.2 * ts)
    as_out[...] = a_s
    ad_out[...] = a_d
    ws_out[...] = jnp.exp(a_self - M)
    m_out[...] = jnp.full((L,), M)


def _tc_alphas(h, att_l, att_r):
    return pl.pallas_call(
        _alpha_body,
        out_shape=(
            jax.ShapeDtypeStruct((NPAD,), jnp.float32),
            jax.ShapeDtypeStruct((NPAD,), jnp.float32),
            jax.ShapeDtypeStruct((NPAD,), jnp.float32),
            jax.ShapeDtypeStruct((L,), jnp.float32),
        ),
    )(h, att_l, att_r)


def _fin_relu_body(acc_ref, den_ref, ws_ref, b_ref, o_ref):
    f = acc_ref[0] + acc_ref[1]
    den = jnp.sum(den_ref[...], axis=0) + ws_ref[...]
    o = f / (den + 1e-16)[:, None] + b_ref[...][None, :]
    o_ref[...] = jnp.maximum(o, 0.0)


def _tc_finalize_relu(acc, den, ws, bias):
    return pl.pallas_call(
        _fin_relu_body,
        out_shape=jax.ShapeDtypeStruct((NPAD, C), jnp.float32),
    )(acc, den, ws, bias)


def _fin_mm_body(acc_ref, den_ref, ws_ref, b_ref, w_ref, b2_ref, o_ref):
    f = acc_ref[0] + acc_ref[1]
    den = jnp.sum(den_ref[...], axis=0) + ws_ref[...]
    o = f / (den + 1e-16)[:, None] + b_ref[...][None, :]
    o_ref[...] = jnp.dot(o, w_ref[...],
                         preferred_element_type=jnp.float32) + b2_ref[...][None, :]


def _tc_finalize_matmul(acc, den, ws, bias, w2, b2):
    return pl.pallas_call(
        _fin_mm_body,
        out_shape=jax.ShapeDtypeStruct((NPAD, C), jnp.float32),
    )(acc, den, ws, bias, w2, b2)


NBUF = 2
EPW = K * NCHUNK


def _stats_body(h_hbm, src_hbm, dst_hbm, out_hbm,
                idx_sv, idx_dv,
                bufS0, bufS1, bufD0, bufD1,
                accA, accB, semS0, semS1, semD0, semD1):
    cid = lax.axis_index("c")
    sid = lax.axis_index("s")
    wid = cid * NS + sid
    base = wid * EPW

    bufS = [bufS0, bufS1]
    bufD = [bufD0, bufD1]
    semS = [semS0, semS1]
    semD = [semD0, semD1]

    pltpu.sync_copy(src_hbm.at[pl.ds(base, EPW)], idx_sv)
    pltpu.sync_copy(dst_hbm.at[pl.ds(base, EPW)], idx_dv)

    zeros = jnp.zeros((L,), jnp.float32)
    for j in range(C // L):
        accA[pl.ds(L * j, L)] = zeros
        accB[pl.ds(L * j, L)] = zeros

    def _fire(c, b):
        sl = pl.ds(c * K, K)
        pltpu.async_copy(h_hbm.at[idx_sv.at[sl]], bufS[b], semS[b])
        pltpu.async_copy(h_hbm.at[idx_dv.at[sl]], bufD[b], semD[b])

    for b in range(NBUF):
        _fire(b, b)

    @pl.loop(0, NCHUNK, step=NBUF)
    def _chunk(k0):
        for b in range(NBUF):
            k = k0 + b
            pltpu.make_async_copy(h_hbm.at[idx_sv.at[pl.ds(0, K)]],
                                  bufS[b], semS[b]).wait()
            pltpu.make_async_copy(h_hbm.at[idx_dv.at[pl.ds(0, K)]],
                                  bufD[b], semD[b]).wait()

            @pl.loop(0, K)
            def _edge(e):
                for j in range(C // L):
                    sl = pl.ds(L * j, L)
                    s = bufS[b][e, sl]
                    t = bufD[b][e, sl]
                    df = s - t
                    d2 = df * df
                    plsc.addupdate(accA.at[sl], d2)
                    plsc.addupdate(accB.at[sl], d2 * d2)

            _fire(lax.rem(k + NBUF, NCHUNK), b)

    for b in range(NBUF):
        pltpu.make_async_copy(h_hbm.at[idx_sv.at[pl.ds(0, K)]],
                              bufS[b], semS[b]).wait()
        pltpu.make_async_copy(h_hbm.at[idx_dv.at[pl.ds(0, K)]],
                              bufD[b], semD[b]).wait()

    pltpu.sync_copy(accA, out_hbm.at[wid, 0])
    pltpu.sync_copy(accB, out_hbm.at[wid, 1])


_stats_kfn = None


def _sc_stats(h, src, dst):
    global _stats_kfn
    if _stats_kfn is None:
        _stats_kfn = _make_stats_kfn()
    return _stats_kfn(h, src, dst)


def _make_stats_kfn():
    return pl.kernel(
        _stats_body,
        mesh=_mesh,
        out_type=jax.ShapeDtypeStruct((NC * NS, 2, C), jnp.float32),
        scratch_types=[
            pltpu.VMEM((EPW,), jnp.int32),
            pltpu.VMEM((EPW,), jnp.int32),
            pltpu.VMEM((K, C), jnp.float32),
            pltpu.VMEM((K, C), jnp.float32),
            pltpu.VMEM((K, C), jnp.float32),
            pltpu.VMEM((K, C), jnp.float32),
            pltpu.VMEM((C,), jnp.float32),
            pltpu.VMEM((C,), jnp.float32),
            pltpu.SemaphoreType.DMA,
            pltpu.SemaphoreType.DMA,
            pltpu.SemaphoreType.DMA,
            pltpu.SemaphoreType.DMA,
        ],
        compiler_params=_cp,
    )


def _agg_body(h_hbm, pack_hbm, as_hbm, ad_hbm, ws_hbm, m_hbm,
              out_hbm, den_hbm,
              alpha_s, alpha_d, idx0, idx1, bufS0, bufS1, wrow,
              wsbuf, m_vmem, denv, acc_sh,
              semI0, semI1, semG0, semG1):
    cid = lax.axis_index("c")
    sid = lax.axis_index("s")
    wid = cid * NS + sid
    cbase = wid * NCHUNK_A
    lane = lax.iota(jnp.int32, L)
    cvec = jnp.full((L,), cid)

    idx2 = [idx0, idx1]
    bufS = [bufS0, bufS1]
    semI = [semI0, semI1]
    semG = [semG0, semG1]

    pltpu.sync_copy(pack_hbm.at[cbase], idx2[0])
    pltpu.async_copy(pack_hbm.at[cbase + 1], idx2[1], semI[1])
    pltpu.async_copy(h_hbm.at[idx2[0].at[0]], bufS[0], semG[0])

    pltpu.sync_copy(as_hbm, alpha_s)
    pltpu.sync_copy(ad_hbm, alpha_d)
    pltpu.sync_copy(m_hbm, m_vmem)
    m = m_vmem[...]

    zeros = jnp.zeros((L,), jnp.float32)

    @pl.loop(0, NPAD // L)
    def _zden(i):
        denv[pl.ds(i * L, L)] = zeros

    row0 = sid * (NPAD // NS)

    @pl.loop(0, (NPAD // NS) // KA)
    def _init(cb):
        r0 = row0 + cb * KA
        pltpu.sync_copy(h_hbm.at[pl.ds(r0, KA)], bufS1)
        pltpu.sync_copy(ws_hbm.at[pl.ds(r0, KA)], wsbuf)

        @pl.loop(0, KA // L)
        def _grp(g):
            wv = wsbuf[pl.ds(L * g, L)]
            wv = jnp.where(cvec == 0, wv, 0.0)
            for e in range(L):
                ws = jnp.sum(jnp.where(lane == e, wv, 0.0))
                wb = jnp.full((L,), ws)
                r = L * g + e
                for j in range(C // L):
                    sl = pl.ds(L * j, L)
                    wrow[r, sl] = bufS1[r, sl] * wb

        pltpu.sync_copy(wrow, acc_sh.at[pl.ds(r0, KA)])

    plsc.subcore_barrier()

    @pl.loop(0, NCHUNK_A, step=NBUF)
    def _chunk(k0):
        for b in range(NBUF):
            k = k0 + b
            bn = (b + 1) % NBUF
            pltpu.make_async_copy(h_hbm.at[idx2[b].at[0]],
                                  bufS[b], semG[b]).wait()
            @pl.loop(0, KA // L)
            def _grp(g):
                sl_g = pl.ds(L * g, L)
                is_v = idx2[b][0, sl_g]
                id_v = idx2[b][1, sl_g]
                as_v = plsc.load_gather(alpha_s, [is_v])
                ad_v = plsc.load_gather(alpha_d, [id_v])
                t = as_v + ad_v
                a = 0.1 * jnp.where(t >= 0, t, 0.2 * t)
                w = jnp.exp(a - m)
                w = jnp.where(is_v != id_v, w, 0.0)
                for e in range(L):
                    ws = jnp.sum(jnp.where(lane == e, w, 0.0))
                    wb = jnp.full((L,), ws)
                    r = L * g + e
                    for j in range(C // L):
                        sl = pl.ds(L * j, L)
                        wrow[r, sl] = bufS[b][r, sl] * wb
                for e in range(L):
                    plsc.addupdate_scatter(denv, [id_v], w, mask=lane == e)

            pltpu.sync_copy(wrow, acc_sh.at[idx2[b].at[1]], add=True)
            pltpu.async_copy(pack_hbm.at[cbase + lax.rem(k + 2, NCHUNK_A)],
                             idx2[b], semI[b])
            pltpu.make_async_copy(pack_hbm.at[cbase], idx2[bn],
                                  semI[bn]).wait()
            pltpu.async_copy(h_hbm.at[idx2[bn].at[0]], bufS[bn], semG[bn])

    pltpu.make_async_copy(pack_hbm.at[cbase], idx2[1], semI[1]).wait()
    pltpu.make_async_copy(h_hbm.at[idx2[0].at[0]], bufS[0], semG[0]).wait()

    plsc.subcore_barrier()
    pltpu.sync_copy(acc_sh.at[pl.ds(row0, NPAD // NS)],
                    out_hbm.at[cid, pl.ds(row0, NPAD // NS)])
    pltpu.sync_copy(denv, den_hbm.at[wid])


_agg_kfn = None


def _sc_aggregate(h, pack, a_s, a_d, w_self, M):
    global _agg_kfn
    if _agg_kfn is None:
        _agg_kfn = _make_agg_kfn()
    return _agg_kfn(h, pack, a_s, a_d, w_self, M)


def _make_agg_kfn():
    return pl.kernel(
        _agg_body,
        mesh=_mesh,
        out_type=(
            jax.ShapeDtypeStruct((NC, NPAD, C), jnp.float32),
            jax.ShapeDtypeStruct((NC * NS, NPAD), jnp.float32),
        ),
        scratch_types=[
            pltpu.VMEM((NPAD,), jnp.float32),
            pltpu.VMEM((NPAD,), jnp.float32),
            pltpu.VMEM((2, KA), jnp.int32),
            pltpu.VMEM((2, KA), jnp.int32),
            pltpu.VMEM((KA, C), jnp.float32),
            pltpu.VMEM((KA, C), jnp.float32),
            pltpu.VMEM((KA, C), jnp.float32),
            pltpu.VMEM((KA,), jnp.float32),
            pltpu.VMEM((L,), jnp.float32),
            pltpu.VMEM((NPAD,), jnp.float32),
            pltpu.VMEM_SHARED((NPAD, C), jnp.float32),
            pltpu.SemaphoreType.DMA,
            pltpu.SemaphoreType.DMA,
            pltpu.SemaphoreType.DMA,
            pltpu.SemaphoreType.DMA,
        ],
        compiler_params=_cp,
    )


def _stats_to_att(parts, sW, sb, dW, db, tq):
    s2 = jnp.sum(parts[:, 0, :], axis=0)
    s4 = jnp.sum(parts[:, 1, :], axis=0)
    m1 = s2 / E
    var = jnp.maximum(s4 - s2 * s2 / E, 0.0) / (E - 1)
    sd = jnp.sqrt(var)
    m2 = sd + 1e-05
    k3 = (m1 * m1 * m1) / (m2 * m2 * m2)
    k4 = (m1 * m1 * m1 * m1) / (m2 * m2 * m2 * m2)
    S = jnp.stack([m1, sd, k3, k4])
    S = jnp.where(jnp.isnan(S), 0.0, S)
    S = jnp.tanh(S)
    nrm = jnp.linalg.norm(S, axis=1, keepdims=True)
    S = (S / jnp.maximum(nrm, 1e-12)).T
    att_l = (S @ sW + sb) @ tq
    att_r = (S @ dW + db) @ tq
    return att_l, att_r


def _spa_layer(h, src, dst, pack, sW, sb, dW, db, tq):
    parts = _sc_stats(h, src, dst)
    att_l, att_r = _stats_to_att(parts, sW, sb, dW, db, tq)
    a_s, a_d, w_self, M = _tc_alphas(h, att_l, att_r)
    acc, den = _sc_aggregate(h, pack, a_s, a_d, w_self, M)
    return acc, den, w_self


def kernel(x, edge_index, fc0_W, fc0_b, g0_sW, g0_sb, g0_dW, g0_db, g0_tq,
           g0_bias, g2_sW, g2_sb, g2_dW, g2_db, g2_tq, g2_bias, fc2_W, fc2_b):
    xp = jnp.concatenate([x, jnp.zeros((NPAD - N, C), jnp.float32)], axis=0)
    src = jnp.concatenate(
        [edge_index[0], jnp.zeros((EPAD - E,), edge_index.dtype)])
    dst = jnp.concatenate(
        [edge_index[1], jnp.zeros((EPAD - E,), edge_index.dtype)])
    pack = jnp.stack([src.reshape(EPAD // KA, KA),
                      dst.reshape(EPAD // KA, KA)], axis=1)

    h = _tc_matmul(xp, fc0_W, fc0_b)
    acc0, den0, ws0 = _spa_layer(h, src, dst, pack,
                                 g0_sW, g0_sb, g0_dW, g0_db, g0_tq)
    h2 = _tc_finalize_relu(acc0, den0, ws0, g0_bias)
    acc1, den1, ws1 = _spa_layer(h2, src, dst, pack,
                                 g2_sW, g2_sb, g2_dW, g2_db, g2_tq)
    out = _tc_finalize_matmul(acc1, den1, ws1, g2_bias, fc2_W, fc2_b)
    return out[:N]
